# Initial kernel scaffold; baseline (speedup 1.0000x reference)
import jax
import jax.numpy as jnp
from jax.experimental import pallas as pl


def kernel(x, edge_index, edge_weight, weight, gamma, beta):
    raise NotImplementedError("write your pallas kernel here")


# speedup vs baseline: 2.3582x; 2.3582x over previous
import jax
import jax.numpy as jnp
from jax import lax
from jax.experimental import pallas as pl
from jax.experimental.pallas import tpu as pltpu
from jax.experimental.pallas import tpu_sc as plsc

V = 10000
FIN = 128
FOUT = 128
KCH = 3
E = 320000
NC = 2
NS = 16
LN = 16
C = 128
ET = 10240
EP = ET * NS * NC
NCHUNK = ET // C
RPT = 624
REM = V - RPT * NS


def _sc_body(table, srcp, dstp, wp, zrows, po, acc, idxb, dstb, wvb, rows):
    c = lax.axis_index("c")
    s = lax.axis_index("s")
    ebase = (c * NS + s) * ET

    pltpu.sync_copy(zrows.at[pl.ds(0, RPT)], acc.at[pl.ds(s * RPT, RPT)])

    @pl.when(s == NS - 1)
    def _():
        pltpu.sync_copy(zrows.at[pl.ds(RPT, REM)],
                        acc.at[pl.ds(NS * RPT, REM)])

    plsc.subcore_barrier()

    def multiply_chunk():
        def g_body(g, carry):
            wvec = wvb[pl.ds(g * LN, LN)]
            for l in range(LN):
                wl = wvec.at[jnp.full((LN,), l, jnp.int32)].get(
                    mode="promise_in_bounds")
                e = g * LN + l
                for j in range(FIN // LN):
                    rows[e, pl.ds(j * LN, LN)] = rows[e, pl.ds(j * LN, LN)] * wl
            return carry
        lax.fori_loop(0, C // LN, g_body, 0)

    def lap_chunk(k, carry):
        base = ebase + k * C
        pltpu.sync_copy(srcp.at[pl.ds(base, C)], idxb)
        pltpu.sync_copy(dstp.at[pl.ds(base, C)], dstb)
        pltpu.sync_copy(wp.at[pl.ds(base, C)], wvb)
        pltpu.sync_copy(table.at[idxb], rows)
        multiply_chunk()
        pltpu.sync_copy(rows, acc.at[dstb], add=True)
        return carry

    lax.fori_loop(0, NCHUNK, lap_chunk, 0)
    plsc.subcore_barrier()

    pltpu.sync_copy(acc.at[pl.ds(s * RPT, RPT)],
                    po.at[pl.ds(c * V + s * RPT, RPT)])

    @pl.when(s == NS - 1)
    def _():
        pltpu.sync_copy(acc.at[pl.ds(NS * RPT, REM)],
                        po.at[pl.ds(c * V + NS * RPT, REM)])


_sc_mesh = plsc.VectorSubcoreMesh(
    core_axis_name="c", subcore_axis_name="s", num_cores=NC, num_subcores=NS)

_sc_call = pl.kernel(
    _sc_body,
    out_type=[jax.ShapeDtypeStruct((NC * V, FIN), jnp.float32)],
    mesh=_sc_mesh,
    scratch_types=[
        pltpu.VMEM_SHARED((V, FIN), jnp.float32),
        pltpu.VMEM((C,), jnp.int32),
        pltpu.VMEM((C,), jnp.int32),
        pltpu.VMEM((C,), jnp.float32),
        pltpu.VMEM((C, FIN), jnp.float32),
    ],
)


def _sum_body(pr, outr):
    def body(b, carry):
        r0 = b * 1000
        outr[pl.ds(r0, 1000), :] = (pr[pl.ds(r0, 1000), :]
                                    + pr[pl.ds(V + r0, 1000), :])
        return carry
    lax.fori_loop(0, V // 1000, body, 0)


_sum_call = pl.pallas_call(
    _sum_body,
    out_shape=jax.ShapeDtypeStruct((V, FIN), jnp.float32),
)


BLK = 500
NB = V // BLK
PBLK = BLK // 4


def _tc_body(x0r, x1r, qr, war, wbr, wcr, gr, br, outr, ys):
    def mm(b, carry):
        sm, sq = carry
        r0 = b * BLK
        yb = jnp.dot(x0r[pl.ds(r0, BLK), :], war[...],
                     preferred_element_type=jnp.float32)
        yb = yb + jnp.dot(x1r[pl.ds(r0, BLK), :], wbr[...],
                          preferred_element_type=jnp.float32)
        yb = yb + jnp.dot(qr[pl.ds(r0, BLK), :] + qr[pl.ds(V + r0, BLK), :],
                          wcr[...], preferred_element_type=jnp.float32)
        ys[pl.ds(r0, BLK), :] = yb
        sm = sm + jnp.sum(yb, axis=0, keepdims=True)
        sq = sq + jnp.sum(yb * yb, axis=0, keepdims=True)
        return sm, sq

    zero = jnp.zeros((1, FOUT), jnp.float32)
    sm, sq = lax.fori_loop(0, NB, mm, (zero, zero))
    mean = sm / float(V)
    var = sq / float(V) - mean * mean
    scale = gr[...] * lax.rsqrt(var + 1e-5)
    shift = br[...] - mean * scale

    def norm(b, carry):
        yb = ys[pl.ds(b * BLK, BLK), :]
        yn = jnp.maximum(yb * scale + shift, 0.0)
        pooled = jnp.mean(yn.reshape(PBLK, 4, FOUT), axis=1)
        outr[pl.ds(b * PBLK, PBLK), :] = pooled
        return carry

    lax.fori_loop(0, NB, norm, 0)


_tc_call = pl.pallas_call(
    _tc_body,
    out_shape=jax.ShapeDtypeStruct((V // 4, FOUT), jnp.float32),
    scratch_shapes=[pltpu.VMEM((V, FOUT), jnp.float32)],
)


def kernel(x, edge_index, edge_weight, weight, gamma, beta):
    x0 = x[0]
    src = edge_index[0]
    dst = edge_index[1]
    pad = EP - E
    srcp = jnp.concatenate([src, jnp.zeros((pad,), jnp.int32)])
    dstp = jnp.concatenate([dst, jnp.zeros((pad,), jnp.int32)])
    wp = jnp.concatenate([edge_weight, jnp.zeros((pad,), jnp.float32)])
    zrows = jnp.zeros((RPT + REM, FIN), jnp.float32)

    w0 = weight[0::KCH]
    w1 = weight[1::KCH]
    w2 = weight[2::KCH]
    wa = w0 - w2
    wb = w1
    wc = 2.0 * w2

    (p,) = _sc_call(x0, srcp, dstp, wp, zrows)
    x1 = _sum_call(p)
    (q,) = _sc_call(x1, srcp, dstp, wp, zrows)
    out = _tc_call(x0, x1, q, wa, wb, wc,
                   gamma.reshape(1, FOUT), beta.reshape(1, FOUT))
    return out[None]


# speedup vs baseline: 3.1539x; 1.3374x over previous
import jax
import jax.numpy as jnp
from jax import lax
from jax.experimental import pallas as pl
from jax.experimental.pallas import tpu as pltpu
from jax.experimental.pallas import tpu_sc as plsc

V = 10000
FIN = 128
FOUT = 128
KCH = 3
E = 320000
NC = 2
NS = 16
LN = 16
C = 128
NCHUNK = 80
SUP = 8
NSUP = NCHUNK // SUP
ET = NCHUNK * C
EP = ET * NS * NC
RPT = 624
REM = V - RPT * NS


def _sc_body(table, edata, wdata, zrows, po,
             acc, eb0, eb1, wb0, wb1, rows0, rows1, gs0, gs1):
    c = lax.axis_index("c")
    s = lax.axis_index("s")
    tid = c * NS + s
    ebufs = (eb0, eb1)
    wbufs = (wb0, wb1)
    rows = (rows0, rows1)
    gsem = (gs0, gs1)

    pltpu.sync_copy(zrows.at[pl.ds(0, RPT)], acc.at[pl.ds(s * RPT, RPT)])

    @pl.when(s == NS - 1)
    def _():
        pltpu.sync_copy(zrows.at[pl.ds(RPT, REM)],
                        acc.at[pl.ds(NS * RPT, REM)])

    pltpu.sync_copy(edata.at[tid, pl.ds(0, SUP)], eb0)
    pltpu.sync_copy(wdata.at[tid, pl.ds(0, SUP)], wb0)
    plsc.subcore_barrier()

    def gather_start(sb, lc, x):
        pltpu.async_copy(table.at[ebufs[sb].at[lc, 0]], rows[x], gsem[x])

    def gather_wait(x):
        pltpu.make_async_copy(table.at[eb0.at[0, 0]], rows[x],
                              gsem[x]).wait()

    def load_sync(sb, si):
        pltpu.sync_copy(edata.at[tid, pl.ds(si * SUP, SUP)], ebufs[sb])
        pltpu.sync_copy(wdata.at[tid, pl.ds(si * SUP, SUP)], wbufs[sb])

    def multiply(sb, lc, x):
        def g_body(g, carry):
            wvec = wbufs[sb][lc, pl.ds(g * LN, LN)]
            for l in range(LN):
                wl = wvec.at[jnp.full((LN,), l, jnp.int32)].get(
                    mode="promise_in_bounds")
                e = g * LN + l
                for j in range(FIN // LN):
                    rows[x][e, pl.ds(j * LN, LN)] = (
                        rows[x][e, pl.ds(j * LN, LN)] * wl)
            return carry
        lax.fori_loop(0, C // LN, g_body, 0)

    gather_start(0, 0, 0)

    def super_pair(u, carry):
        for sb in range(2):
            si = u * 2 + sb
            base = si * SUP
            for lc in range(SUP):
                d = lc % 2
                gather_wait(d)
                if lc < SUP - 1:
                    gather_start(sb, lc + 1, 1 - d)
                elif sb == 0:
                    load_sync(1, si + 1)
                    gather_start(1, 0, 1 - d)
                else:
                    @pl.when(si + 1 < NSUP)
                    def _():
                        load_sync(0, si + 1)
                        gather_start(0, 0, 1 - d)
                multiply(sb, lc, d)
                pltpu.sync_copy(rows[d], acc.at[ebufs[sb].at[lc, 1]],
                                add=True)
        return carry

    lax.fori_loop(0, NSUP // 2, super_pair, 0)
    plsc.subcore_barrier()

    pltpu.sync_copy(acc.at[pl.ds(s * RPT, RPT)],
                    po.at[pl.ds(c * V + s * RPT, RPT)])

    @pl.when(s == NS - 1)
    def _():
        pltpu.sync_copy(acc.at[pl.ds(NS * RPT, REM)],
                        po.at[pl.ds(c * V + NS * RPT, REM)])


_sc_mesh = plsc.VectorSubcoreMesh(
    core_axis_name="c", subcore_axis_name="s", num_cores=NC, num_subcores=NS)

_sc_call = pl.kernel(
    _sc_body,
    out_type=[jax.ShapeDtypeStruct((NC * V, FIN), jnp.float32)],
    mesh=_sc_mesh,
    scratch_types=[
        pltpu.VMEM_SHARED((V, FIN), jnp.float32),
        pltpu.VMEM((SUP, 2, C), jnp.int32),
        pltpu.VMEM((SUP, 2, C), jnp.int32),
        pltpu.VMEM((SUP, C), jnp.float32),
        pltpu.VMEM((SUP, C), jnp.float32),
        pltpu.VMEM((C, FIN), jnp.float32),
        pltpu.VMEM((C, FIN), jnp.float32),
        pltpu.SemaphoreType.DMA,
        pltpu.SemaphoreType.DMA,
    ],
)


def _sum_body(pr, outr):
    def body(b, carry):
        r0 = b * 1000
        outr[pl.ds(r0, 1000), :] = (pr[pl.ds(r0, 1000), :]
                                    + pr[pl.ds(V + r0, 1000), :])
        return carry
    lax.fori_loop(0, V // 1000, body, 0)


_sum_call = pl.pallas_call(
    _sum_body,
    out_shape=jax.ShapeDtypeStruct((V, FIN), jnp.float32),
)


BLK = 500
NB = V // BLK
PBLK = BLK // 4


def _tc_body(x0r, x1r, qr, war, wbr, wcr, gr, br, outr, ys):
    def mm(b, carry):
        sm, sq = carry
        r0 = b * BLK
        yb = jnp.dot(x0r[pl.ds(r0, BLK), :], war[...],
                     preferred_element_type=jnp.float32)
        yb = yb + jnp.dot(x1r[pl.ds(r0, BLK), :], wbr[...],
                          preferred_element_type=jnp.float32)
        yb = yb + jnp.dot(qr[pl.ds(r0, BLK), :] + qr[pl.ds(V + r0, BLK), :],
                          wcr[...], preferred_element_type=jnp.float32)
        ys[pl.ds(r0, BLK), :] = yb
        sm = sm + jnp.sum(yb, axis=0, keepdims=True)
        sq = sq + jnp.sum(yb * yb, axis=0, keepdims=True)
        return sm, sq

    zero = jnp.zeros((1, FOUT), jnp.float32)
    sm, sq = lax.fori_loop(0, NB, mm, (zero, zero))
    mean = sm / float(V)
    var = sq / float(V) - mean * mean
    scale = gr[...] * lax.rsqrt(var + 1e-5)
    shift = br[...] - mean * scale

    def norm(b, carry):
        yb = ys[pl.ds(b * BLK, BLK), :]
        yn = jnp.maximum(yb * scale + shift, 0.0)
        pooled = jnp.mean(yn.reshape(PBLK, 4, FOUT), axis=1)
        outr[pl.ds(b * PBLK, PBLK), :] = pooled
        return carry

    lax.fori_loop(0, NB, norm, 0)


_tc_call = pl.pallas_call(
    _tc_body,
    out_shape=jax.ShapeDtypeStruct((V // 4, FOUT), jnp.float32),
    scratch_shapes=[pltpu.VMEM((V, FOUT), jnp.float32)],
)


def kernel(x, edge_index, edge_weight, weight, gamma, beta):
    x0 = x[0]
    src = edge_index[0]
    dst = edge_index[1]
    pad = EP - E
    srcp = jnp.concatenate([src, jnp.zeros((pad,), jnp.int32)])
    dstp = jnp.concatenate([dst, jnp.zeros((pad,), jnp.int32)])
    wp = jnp.concatenate([edge_weight, jnp.zeros((pad,), jnp.float32)])
    edata = jnp.stack([srcp.reshape(NC * NS, NCHUNK, C),
                       dstp.reshape(NC * NS, NCHUNK, C)], axis=2)
    wdata = wp.reshape(NC * NS, NCHUNK, C)
    zrows = jnp.zeros((RPT + REM, FIN), jnp.float32)

    w0 = weight[0::KCH]
    w1 = weight[1::KCH]
    w2 = weight[2::KCH]
    wa = w0 - w2
    wb = w1
    wc = 2.0 * w2

    (p,) = _sc_call(x0, edata, wdata, zrows)
    x1 = _sum_call(p)
    (q,) = _sc_call(x1, edata, wdata, zrows)
    out = _tc_call(x0, x1, q, wa, wb, wc,
                   gamma.reshape(1, FOUT), beta.reshape(1, FOUT))
    return out[None]


# speedup vs baseline: 3.1561x; 1.0007x over previous
import jax
import jax.numpy as jnp
from jax import lax
from jax.experimental import pallas as pl
from jax.experimental.pallas import tpu as pltpu
from jax.experimental.pallas import tpu_sc as plsc

V = 10000
FIN = 128
FOUT = 128
KCH = 3
E = 320000
NC = 2
NS = 16
LN = 16
C = 128
NCHUNK = 80
SUP = 8
NSUP = NCHUNK // SUP
ET = NCHUNK * C
EP = ET * NS * NC
RPT = 624
REM = V - RPT * NS


def _sc_body(table, edata, wdata, zrows, po,
             acc, eb0, eb1, wb0, wb1, rows0, rows1, gs0, gs1, ss0, ss1):
    c = lax.axis_index("c")
    s = lax.axis_index("s")
    tid = c * NS + s
    ebufs = (eb0, eb1)
    wbufs = (wb0, wb1)
    rows = (rows0, rows1)
    gsem = (gs0, gs1)
    ssem = (ss0, ss1)

    pltpu.sync_copy(zrows.at[pl.ds(0, RPT)], acc.at[pl.ds(s * RPT, RPT)])

    @pl.when(s == NS - 1)
    def _():
        pltpu.sync_copy(zrows.at[pl.ds(RPT, REM)],
                        acc.at[pl.ds(NS * RPT, REM)])

    pltpu.sync_copy(edata.at[tid, pl.ds(0, SUP)], eb0)
    pltpu.sync_copy(wdata.at[tid, pl.ds(0, SUP)], wb0)
    plsc.subcore_barrier()

    def gather_start(sb, lc, x):
        pltpu.async_copy(table.at[ebufs[sb].at[lc, 0]], rows[x], gsem[x])

    def gather_wait(x):
        pltpu.make_async_copy(table.at[eb0.at[0, 0]], rows[x],
                              gsem[x]).wait()

    def load_sync(sb, si):
        pltpu.sync_copy(edata.at[tid, pl.ds(si * SUP, SUP)], ebufs[sb])
        pltpu.sync_copy(wdata.at[tid, pl.ds(si * SUP, SUP)], wbufs[sb])

    def scatter_start(sb, lc, x):
        pltpu.async_copy(rows[x], acc.at[ebufs[sb].at[lc, 1]], ssem[x],
                         add=True)

    def scatter_wait(x):
        pltpu.make_async_copy(rows[x], acc.at[eb0.at[0, 1]], ssem[x]).wait()

    def multiply(sb, lc, x):
        def g_body(g, carry):
            wvec = wbufs[sb][lc, pl.ds(g * LN, LN)]
            for l in range(LN):
                wl = wvec.at[jnp.full((LN,), l, jnp.int32)].get(
                    mode="promise_in_bounds")
                e = g * LN + l
                for j in range(FIN // LN):
                    rows[x][e, pl.ds(j * LN, LN)] = (
                        rows[x][e, pl.ds(j * LN, LN)] * wl)
            return carry
        lax.fori_loop(0, C // LN, g_body, 0)

    gather_start(0, 0, 0)

    def super_pair(u, carry):
        for sb in range(2):
            si = u * 2 + sb
            base = si * SUP
            for lc in range(SUP):
                d = lc % 2
                ch = base + lc
                gather_wait(d)
                if lc < SUP - 1:
                    @pl.when(ch >= 1)
                    def _():
                        scatter_wait(1 - d)
                    gather_start(sb, lc + 1, 1 - d)
                elif sb == 0:
                    scatter_wait(1 - d)
                    load_sync(1, si + 1)
                    gather_start(1, 0, 1 - d)
                else:
                    @pl.when(si + 1 < NSUP)
                    def _():
                        scatter_wait(1 - d)
                        load_sync(0, si + 1)
                        gather_start(0, 0, 1 - d)
                multiply(sb, lc, d)
                scatter_start(sb, lc, d)
        return carry

    lax.fori_loop(0, NSUP // 2, super_pair, 0)
    scatter_wait(0)
    scatter_wait(1)
    plsc.subcore_barrier()

    pltpu.sync_copy(acc.at[pl.ds(s * RPT, RPT)],
                    po.at[pl.ds(c * V + s * RPT, RPT)])

    @pl.when(s == NS - 1)
    def _():
        pltpu.sync_copy(acc.at[pl.ds(NS * RPT, REM)],
                        po.at[pl.ds(c * V + NS * RPT, REM)])


_sc_mesh = plsc.VectorSubcoreMesh(
    core_axis_name="c", subcore_axis_name="s", num_cores=NC, num_subcores=NS)

_sc_call = pl.kernel(
    _sc_body,
    out_type=[jax.ShapeDtypeStruct((NC * V, FIN), jnp.float32)],
    mesh=_sc_mesh,
    scratch_types=[
        pltpu.VMEM_SHARED((V, FIN), jnp.float32),
        pltpu.VMEM((SUP, 2, C), jnp.int32),
        pltpu.VMEM((SUP, 2, C), jnp.int32),
        pltpu.VMEM((SUP, C), jnp.float32),
        pltpu.VMEM((SUP, C), jnp.float32),
        pltpu.VMEM((C, FIN), jnp.float32),
        pltpu.VMEM((C, FIN), jnp.float32),
        pltpu.SemaphoreType.DMA,
        pltpu.SemaphoreType.DMA,
        pltpu.SemaphoreType.DMA,
        pltpu.SemaphoreType.DMA,
    ],
)


def _sum_body(pr, outr):
    def body(b, carry):
        r0 = b * 1000
        outr[pl.ds(r0, 1000), :] = (pr[pl.ds(r0, 1000), :]
                                    + pr[pl.ds(V + r0, 1000), :])
        return carry
    lax.fori_loop(0, V // 1000, body, 0)


_sum_call = pl.pallas_call(
    _sum_body,
    out_shape=jax.ShapeDtypeStruct((V, FIN), jnp.float32),
)


BLK = 500
NB = V // BLK
PBLK = BLK // 4


def _tc_body(x0r, x1r, qr, war, wbr, wcr, gr, br, outr, ys):
    def mm(b, carry):
        sm, sq = carry
        r0 = b * BLK
        yb = jnp.dot(x0r[pl.ds(r0, BLK), :], war[...],
                     preferred_element_type=jnp.float32)
        yb = yb + jnp.dot(x1r[pl.ds(r0, BLK), :], wbr[...],
                          preferred_element_type=jnp.float32)
        yb = yb + jnp.dot(qr[pl.ds(r0, BLK), :] + qr[pl.ds(V + r0, BLK), :],
                          wcr[...], preferred_element_type=jnp.float32)
        ys[pl.ds(r0, BLK), :] = yb
        sm = sm + jnp.sum(yb, axis=0, keepdims=True)
        sq = sq + jnp.sum(yb * yb, axis=0, keepdims=True)
        return sm, sq

    zero = jnp.zeros((1, FOUT), jnp.float32)
    sm, sq = lax.fori_loop(0, NB, mm, (zero, zero))
    mean = sm / float(V)
    var = sq / float(V) - mean * mean
    scale = gr[...] * lax.rsqrt(var + 1e-5)
    shift = br[...] - mean * scale

    def norm(b, carry):
        yb = ys[pl.ds(b * BLK, BLK), :]
        yn = jnp.maximum(yb * scale + shift, 0.0)
        pooled = jnp.mean(yn.reshape(PBLK, 4, FOUT), axis=1)
        outr[pl.ds(b * PBLK, PBLK), :] = pooled
        return carry

    lax.fori_loop(0, NB, norm, 0)


_tc_call = pl.pallas_call(
    _tc_body,
    out_shape=jax.ShapeDtypeStruct((V // 4, FOUT), jnp.float32),
    scratch_shapes=[pltpu.VMEM((V, FOUT), jnp.float32)],
)


def kernel(x, edge_index, edge_weight, weight, gamma, beta):
    x0 = x[0]
    src = edge_index[0]
    dst = edge_index[1]
    pad = EP - E
    srcp = jnp.concatenate([src, jnp.zeros((pad,), jnp.int32)])
    dstp = jnp.concatenate([dst, jnp.zeros((pad,), jnp.int32)])
    wp = jnp.concatenate([edge_weight, jnp.zeros((pad,), jnp.float32)])
    edata = jnp.stack([srcp.reshape(NC * NS, NCHUNK, C),
                       dstp.reshape(NC * NS, NCHUNK, C)], axis=2)
    wdata = wp.reshape(NC * NS, NCHUNK, C)
    zrows = jnp.zeros((RPT + REM, FIN), jnp.float32)

    w0 = weight[0::KCH]
    w1 = weight[1::KCH]
    w2 = weight[2::KCH]
    wa = w0 - w2
    wb = w1
    wc = 2.0 * w2

    (p,) = _sc_call(x0, edata, wdata, zrows)
    x1 = _sum_call(p)
    (q,) = _sc_call(x1, edata, wdata, zrows)
    out = _tc_call(x0, x1, q, wa, wb, wc,
                   gamma.reshape(1, FOUT), beta.reshape(1, FOUT))
    return out[None]


# speedup vs baseline: 3.1562x; 1.0000x over previous
import jax
import jax.numpy as jnp
from jax import lax
from jax.experimental import pallas as pl
from jax.experimental.pallas import tpu as pltpu
from jax.experimental.pallas import tpu_sc as plsc

V = 10000
FIN = 128
FOUT = 128
KCH = 3
E = 320000
NC = 2
NS = 16
LN = 16
C = 128
NCHUNK = 80
SUP = 8
NSUP = NCHUNK // SUP
ET = NCHUNK * C
EP = ET * NS * NC
RPT = 624
REM = V - RPT * NS


def _sc_body(table, edata, wdata, zrows, po,
             acc, eb0, eb1, wb0, wb1, rows0, rows1, gs0, gs1, ss0, ss1):
    c = lax.axis_index("c")
    s = lax.axis_index("s")
    tid = c * NS + s
    ebufs = (eb0, eb1)
    wbufs = (wb0, wb1)
    rows = (rows0, rows1)
    gsem = (gs0, gs1)
    ssem = (ss0, ss1)

    pltpu.sync_copy(zrows.at[pl.ds(0, RPT)], acc.at[pl.ds(s * RPT, RPT)])

    @pl.when(s == NS - 1)
    def _():
        pltpu.sync_copy(zrows.at[pl.ds(RPT, REM)],
                        acc.at[pl.ds(NS * RPT, REM)])

    pltpu.sync_copy(edata.at[tid, pl.ds(0, SUP)], eb0)
    pltpu.sync_copy(wdata.at[tid, pl.ds(0, SUP)], wb0)
    plsc.subcore_barrier()

    def gather_start(sb, lc, x):
        pltpu.async_copy(table.at[ebufs[sb].at[lc, 0, pl.ds(0, C // 2)]],
                         rows[x].at[pl.ds(0, C // 2)], gsem[x])
        pltpu.async_copy(table.at[ebufs[sb].at[lc, 0, pl.ds(C // 2, C // 2)]],
                         rows[x].at[pl.ds(C // 2, C // 2)], gsem[x])

    def gather_wait(x):
        pltpu.make_async_copy(table.at[eb0.at[0, 0, pl.ds(0, C // 2)]],
                              rows[x].at[pl.ds(0, C // 2)], gsem[x]).wait()
        pltpu.make_async_copy(table.at[eb0.at[0, 0, pl.ds(C // 2, C // 2)]],
                              rows[x].at[pl.ds(C // 2, C // 2)], gsem[x]).wait()

    def load_sync(sb, si):
        pltpu.sync_copy(edata.at[tid, pl.ds(si * SUP, SUP)], ebufs[sb])
        pltpu.sync_copy(wdata.at[tid, pl.ds(si * SUP, SUP)], wbufs[sb])

    def scatter_start(sb, lc, x):
        pltpu.async_copy(rows[x], acc.at[ebufs[sb].at[lc, 1]], ssem[x],
                         add=True)

    def scatter_wait(x):
        pltpu.make_async_copy(rows[x], acc.at[eb0.at[0, 1]], ssem[x]).wait()

    def multiply(sb, lc, x):
        def g_body(g, carry):
            wvec = wbufs[sb][lc, pl.ds(g * LN, LN)]
            for l in range(LN):
                wl = wvec.at[jnp.full((LN,), l, jnp.int32)].get(
                    mode="promise_in_bounds")
                e = g * LN + l
                for j in range(FIN // LN):
                    rows[x][e, pl.ds(j * LN, LN)] = (
                        rows[x][e, pl.ds(j * LN, LN)] * wl)
            return carry
        lax.fori_loop(0, C // LN, g_body, 0)

    gather_start(0, 0, 0)

    def super_pair(u, carry):
        for sb in range(2):
            si = u * 2 + sb
            base = si * SUP
            for lc in range(SUP):
                d = lc % 2
                ch = base + lc
                gather_wait(d)
                if lc < SUP - 1:
                    @pl.when(ch >= 1)
                    def _():
                        scatter_wait(1 - d)
                    gather_start(sb, lc + 1, 1 - d)
                elif sb == 0:
                    scatter_wait(1 - d)
                    load_sync(1, si + 1)
                    gather_start(1, 0, 1 - d)
                else:
                    @pl.when(si + 1 < NSUP)
                    def _():
                        scatter_wait(1 - d)
                        load_sync(0, si + 1)
                        gather_start(0, 0, 1 - d)
                multiply(sb, lc, d)
                scatter_start(sb, lc, d)
        return carry

    lax.fori_loop(0, NSUP // 2, super_pair, 0)
    scatter_wait(0)
    scatter_wait(1)
    plsc.subcore_barrier()

    pltpu.sync_copy(acc.at[pl.ds(s * RPT, RPT)],
                    po.at[pl.ds(c * V + s * RPT, RPT)])

    @pl.when(s == NS - 1)
    def _():
        pltpu.sync_copy(acc.at[pl.ds(NS * RPT, REM)],
                        po.at[pl.ds(c * V + NS * RPT, REM)])


_sc_mesh = plsc.VectorSubcoreMesh(
    core_axis_name="c", subcore_axis_name="s", num_cores=NC, num_subcores=NS)

_sc_call = pl.kernel(
    _sc_body,
    out_type=[jax.ShapeDtypeStruct((NC * V, FIN), jnp.float32)],
    mesh=_sc_mesh,
    scratch_types=[
        pltpu.VMEM_SHARED((V, FIN), jnp.float32),
        pltpu.VMEM((SUP, 2, C), jnp.int32),
        pltpu.VMEM((SUP, 2, C), jnp.int32),
        pltpu.VMEM((SUP, C), jnp.float32),
        pltpu.VMEM((SUP, C), jnp.float32),
        pltpu.VMEM((C, FIN), jnp.float32),
        pltpu.VMEM((C, FIN), jnp.float32),
        pltpu.SemaphoreType.DMA,
        pltpu.SemaphoreType.DMA,
        pltpu.SemaphoreType.DMA,
        pltpu.SemaphoreType.DMA,
    ],
)


def _sum_body(pr, outr):
    def body(b, carry):
        r0 = b * 1000
        outr[pl.ds(r0, 1000), :] = (pr[pl.ds(r0, 1000), :]
                                    + pr[pl.ds(V + r0, 1000), :])
        return carry
    lax.fori_loop(0, V // 1000, body, 0)


_sum_call = pl.pallas_call(
    _sum_body,
    out_shape=jax.ShapeDtypeStruct((V, FIN), jnp.float32),
)


BLK = 500
NB = V // BLK
PBLK = BLK // 4


def _tc_body(x0r, x1r, qr, war, wbr, wcr, gr, br, outr, ys):
    def mm(b, carry):
        sm, sq = carry
        r0 = b * BLK
        yb = jnp.dot(x0r[pl.ds(r0, BLK), :], war[...],
                     preferred_element_type=jnp.float32)
        yb = yb + jnp.dot(x1r[pl.ds(r0, BLK), :], wbr[...],
                          preferred_element_type=jnp.float32)
        yb = yb + jnp.dot(qr[pl.ds(r0, BLK), :] + qr[pl.ds(V + r0, BLK), :],
                          wcr[...], preferred_element_type=jnp.float32)
        ys[pl.ds(r0, BLK), :] = yb
        sm = sm + jnp.sum(yb, axis=0, keepdims=True)
        sq = sq + jnp.sum(yb * yb, axis=0, keepdims=True)
        return sm, sq

    zero = jnp.zeros((1, FOUT), jnp.float32)
    sm, sq = lax.fori_loop(0, NB, mm, (zero, zero))
    mean = sm / float(V)
    var = sq / float(V) - mean * mean
    scale = gr[...] * lax.rsqrt(var + 1e-5)
    shift = br[...] - mean * scale

    def norm(b, carry):
        yb = ys[pl.ds(b * BLK, BLK), :]
        yn = jnp.maximum(yb * scale + shift, 0.0)
        pooled = jnp.mean(yn.reshape(PBLK, 4, FOUT), axis=1)
        outr[pl.ds(b * PBLK, PBLK), :] = pooled
        return carry

    lax.fori_loop(0, NB, norm, 0)


_tc_call = pl.pallas_call(
    _tc_body,
    out_shape=jax.ShapeDtypeStruct((V // 4, FOUT), jnp.float32),
    scratch_shapes=[pltpu.VMEM((V, FOUT), jnp.float32)],
)


def kernel(x, edge_index, edge_weight, weight, gamma, beta):
    x0 = x[0]
    src = edge_index[0]
    dst = edge_index[1]
    pad = EP - E
    srcp = jnp.concatenate([src, jnp.zeros((pad,), jnp.int32)])
    dstp = jnp.concatenate([dst, jnp.zeros((pad,), jnp.int32)])
    wp = jnp.concatenate([edge_weight, jnp.zeros((pad,), jnp.float32)])
    edata = jnp.stack([srcp.reshape(NC * NS, NCHUNK, C),
                       dstp.reshape(NC * NS, NCHUNK, C)], axis=2)
    wdata = wp.reshape(NC * NS, NCHUNK, C)
    zrows = jnp.zeros((RPT + REM, FIN), jnp.float32)

    w0 = weight[0::KCH]
    w1 = weight[1::KCH]
    w2 = weight[2::KCH]
    wa = w0 - w2
    wb = w1
    wc = 2.0 * w2

    (p,) = _sc_call(x0, edata, wdata, zrows)
    x1 = _sum_call(p)
    (q,) = _sc_call(x1, edata, wdata, zrows)
    out = _tc_call(x0, x1, q, wa, wb, wc,
                   gamma.reshape(1, FOUT), beta.reshape(1, FOUT))
    return out[None]


# speedup vs baseline: 5.3544x; 1.6965x over previous
import jax
import jax.numpy as jnp
from jax import lax
from jax.experimental import pallas as pl
from jax.experimental.pallas import tpu as pltpu
from jax.experimental.pallas import tpu_sc as plsc

V = 10000
FIN = 128
FOUT = 128
KCH = 3
E = 320000
CH = 64
NC = 2
NS = 16
LN = 16
C = 128
NCHUNK = 80
SUP = 8
NSUP = NCHUNK // SUP
ET = NCHUNK * C
EP = ET * NS * NC
RPT = 624
REM = V - RPT * NS


def _sc_body(table2, edata, wdata, zrows, po,
             tab, acc, eb0, eb1, wb0, wb1, rows0, rows1,
             gs0, gs1, ss0, ss1):
    c = lax.axis_index("c")
    s = lax.axis_index("s")
    tid = c * NS + s
    ebufs = (eb0, eb1)
    wbufs = (wb0, wb1)
    rows = (rows0, rows1)
    gsem = (gs0, gs1)
    ssem = (ss0, ss1)

    def gather_start(sb, lc, x):
        pltpu.async_copy(tab.at[ebufs[sb].at[lc, 0]], rows[x], gsem[x])

    def gather_wait(x):
        pltpu.make_async_copy(tab.at[eb0.at[0, 0]], rows[x], gsem[x]).wait()

    def scatter_start(sb, lc, x):
        pltpu.async_copy(rows[x], acc.at[ebufs[sb].at[lc, 1]], ssem[x],
                         add=True)

    def scatter_wait(x):
        pltpu.make_async_copy(rows[x], acc.at[eb0.at[0, 1]], ssem[x]).wait()

    def load_sync(sb, si):
        pltpu.sync_copy(edata.at[tid, pl.ds(si * SUP, SUP)], ebufs[sb])
        pltpu.sync_copy(wdata.at[tid, pl.ds(si * SUP, SUP)], wbufs[sb])

    def multiply(sb, lc, x):
        def g_body(g, carry):
            wvec = wbufs[sb][lc, pl.ds(g * LN, LN)]
            for l in range(LN):
                wl = wvec.at[jnp.full((LN,), l, jnp.int32)].get(
                    mode="promise_in_bounds")
                e = g * LN + l
                for j in range(CH // LN):
                    rows[x][e, pl.ds(j * LN, LN)] = (
                        rows[x][e, pl.ds(j * LN, LN)] * wl)
            return carry
        lax.fori_loop(0, C // LN, g_body, 0)

    def half_body(h, carry):
        pltpu.sync_copy(table2.at[h, pl.ds(s * RPT, RPT)],
                        tab.at[pl.ds(s * RPT, RPT)])
        pltpu.sync_copy(zrows.at[pl.ds(0, RPT)], acc.at[pl.ds(s * RPT, RPT)])

        @pl.when(s == NS - 1)
        def _():
            pltpu.sync_copy(table2.at[h, pl.ds(NS * RPT, REM)],
                            tab.at[pl.ds(NS * RPT, REM)])
            pltpu.sync_copy(zrows.at[pl.ds(RPT, REM)],
                            acc.at[pl.ds(NS * RPT, REM)])

        load_sync(0, 0)
        plsc.subcore_barrier()

        gather_start(0, 0, 0)

        def super_pair(u, carry2):
            for sb in range(2):
                si = u * 2 + sb
                base = si * SUP
                for lc in range(SUP):
                    d = lc % 2
                    ch = base + lc
                    gather_wait(d)
                    if lc < SUP - 1:
                        @pl.when(ch >= 1)
                        def _():
                            scatter_wait(1 - d)
                        gather_start(sb, lc + 1, 1 - d)
                    elif sb == 0:
                        scatter_wait(1 - d)
                        load_sync(1, si + 1)
                        gather_start(1, 0, 1 - d)
                    else:
                        @pl.when(si + 1 < NSUP)
                        def _():
                            scatter_wait(1 - d)
                            load_sync(0, si + 1)
                            gather_start(0, 0, 1 - d)
                    multiply(sb, lc, d)
                    scatter_start(sb, lc, d)
            return carry2

        lax.fori_loop(0, NSUP // 2, super_pair, 0)
        scatter_wait(0)
        scatter_wait(1)
        plsc.subcore_barrier()

        pltpu.sync_copy(acc.at[pl.ds(s * RPT, RPT)],
                        po.at[c, h, pl.ds(s * RPT, RPT)])

        @pl.when(s == NS - 1)
        def _():
            pltpu.sync_copy(acc.at[pl.ds(NS * RPT, REM)],
                            po.at[c, h, pl.ds(NS * RPT, REM)])

        plsc.subcore_barrier()
        return carry

    lax.fori_loop(0, 2, half_body, 0)


_sc_mesh = plsc.VectorSubcoreMesh(
    core_axis_name="c", subcore_axis_name="s", num_cores=NC, num_subcores=NS)

_sc_call = pl.kernel(
    _sc_body,
    out_type=[jax.ShapeDtypeStruct((NC, 2, V, CH), jnp.float32)],
    mesh=_sc_mesh,
    scratch_types=[
        pltpu.VMEM_SHARED((V, CH), jnp.float32),
        pltpu.VMEM_SHARED((V, CH), jnp.float32),
        pltpu.VMEM((SUP, 2, C), jnp.int32),
        pltpu.VMEM((SUP, 2, C), jnp.int32),
        pltpu.VMEM((SUP, C), jnp.float32),
        pltpu.VMEM((SUP, C), jnp.float32),
        pltpu.VMEM((C, CH), jnp.float32),
        pltpu.VMEM((C, CH), jnp.float32),
        pltpu.SemaphoreType.DMA,
        pltpu.SemaphoreType.DMA,
        pltpu.SemaphoreType.DMA,
        pltpu.SemaphoreType.DMA,
    ],
)


def _sum_body(pr, outr):
    def body(b, carry):
        r0 = b * 1000
        for h in range(2):
            outr[h, pl.ds(r0, 1000), :] = (pr[0, h, pl.ds(r0, 1000), :]
                                           + pr[1, h, pl.ds(r0, 1000), :])
        return carry
    lax.fori_loop(0, V // 1000, body, 0)


_sum_call = pl.pallas_call(
    _sum_body,
    out_shape=jax.ShapeDtypeStruct((2, V, CH), jnp.float32),
)


BLK = 500
NB = V // BLK
PBLK = BLK // 4


def _tc_body(x0r, x1r, qr, war, wbr, wcr, gr, br, outr, ys):
    def mm(b, carry):
        sm, sq = carry
        r0 = b * BLK
        yb = jnp.dot(x0r[pl.ds(r0, BLK), :], war[...],
                     preferred_element_type=jnp.float32)
        yb = yb + jnp.dot(x1r[0, pl.ds(r0, BLK), :], wbr[:CH, :],
                          preferred_element_type=jnp.float32)
        yb = yb + jnp.dot(x1r[1, pl.ds(r0, BLK), :], wbr[CH:, :],
                          preferred_element_type=jnp.float32)
        yb = yb + jnp.dot(qr[0, 0, pl.ds(r0, BLK), :]
                          + qr[1, 0, pl.ds(r0, BLK), :], wcr[:CH, :],
                          preferred_element_type=jnp.float32)
        yb = yb + jnp.dot(qr[0, 1, pl.ds(r0, BLK), :]
                          + qr[1, 1, pl.ds(r0, BLK), :], wcr[CH:, :],
                          preferred_element_type=jnp.float32)
        ys[pl.ds(r0, BLK), :] = yb
        sm = sm + jnp.sum(yb, axis=0, keepdims=True)
        sq = sq + jnp.sum(yb * yb, axis=0, keepdims=True)
        return sm, sq

    zero = jnp.zeros((1, FOUT), jnp.float32)
    sm, sq = lax.fori_loop(0, NB, mm, (zero, zero))
    mean = sm / float(V)
    var = sq / float(V) - mean * mean
    scale = gr[...] * lax.rsqrt(var + 1e-5)
    shift = br[...] - mean * scale

    def norm(b, carry):
        yb = ys[pl.ds(b * BLK, BLK), :]
        yn = jnp.maximum(yb * scale + shift, 0.0)
        pooled = jnp.mean(yn.reshape(PBLK, 4, FOUT), axis=1)
        outr[pl.ds(b * PBLK, PBLK), :] = pooled
        return carry

    lax.fori_loop(0, NB, norm, 0)


_tc_call = pl.pallas_call(
    _tc_body,
    out_shape=jax.ShapeDtypeStruct((V // 4, FOUT), jnp.float32),
    scratch_shapes=[pltpu.VMEM((V, FOUT), jnp.float32)],
)


def kernel(x, edge_index, edge_weight, weight, gamma, beta):
    x0 = x[0]
    src = edge_index[0]
    dst = edge_index[1]
    pad = EP - E
    srcp = jnp.concatenate([src, jnp.zeros((pad,), jnp.int32)])
    dstp = jnp.concatenate([dst, jnp.zeros((pad,), jnp.int32)])
    wp = jnp.concatenate([edge_weight, jnp.zeros((pad,), jnp.float32)])
    edata = jnp.stack([srcp.reshape(NC * NS, NCHUNK, C),
                       dstp.reshape(NC * NS, NCHUNK, C)], axis=2)
    wdata = wp.reshape(NC * NS, NCHUNK, C)
    x0h = jnp.stack([x0[:, :CH], x0[:, CH:]])
    zrows = jnp.zeros((RPT + REM, CH), jnp.float32)

    w0 = weight[0::KCH]
    w1 = weight[1::KCH]
    w2 = weight[2::KCH]
    wa = w0 - w2
    wb = w1
    wc = 2.0 * w2

    (p,) = _sc_call(x0h, edata, wdata, zrows)
    x1h = _sum_call(p)
    (q,) = _sc_call(x1h, edata, wdata, zrows)
    out = _tc_call(x0, x1h, q, wa, wb, wc,
                   gamma.reshape(1, FOUT), beta.reshape(1, FOUT))
    return out[None]


# speedup vs baseline: 5.4562x; 1.0190x over previous
import jax
import jax.numpy as jnp
from jax import lax
from jax.experimental import pallas as pl
from jax.experimental.pallas import tpu as pltpu
from jax.experimental.pallas import tpu_sc as plsc

V = 10000
FIN = 128
FOUT = 128
KCH = 3
E = 320000
CH = 64
NC = 2
NS = 16
LN = 16
C = 64
NCHUNK = 160
SUP = 8
NSUP = NCHUNK // SUP
ET = NCHUNK * C
EP = ET * NS * NC
RPT = 624
REM = V - RPT * NS


NRING = 4


def _sc_body(table2, edata, wdata, zrows, po,
             tab, acc, eb0, eb1, wb0, wb1, rows0, rows1, rows2, rows3,
             gs0, gs1, gs2, gs3, ss0, ss1, ss2, ss3):
    c = lax.axis_index("c")
    s = lax.axis_index("s")
    tid = c * NS + s
    ebufs = (eb0, eb1)
    wbufs = (wb0, wb1)
    rows = (rows0, rows1, rows2, rows3)
    gsem = (gs0, gs1, gs2, gs3)
    ssem = (ss0, ss1, ss2, ss3)

    def gather_start(sb, lc, x):
        pltpu.async_copy(tab.at[ebufs[sb].at[lc, 0]], rows[x], gsem[x])

    def gather_wait(x):
        pltpu.make_async_copy(tab.at[eb0.at[0, 0]], rows[x], gsem[x]).wait()

    def scatter_start(sb, lc, x):
        pltpu.async_copy(rows[x], acc.at[ebufs[sb].at[lc, 1]], ssem[x],
                         add=True)

    def scatter_wait(x):
        pltpu.make_async_copy(rows[x], acc.at[eb0.at[0, 1]], ssem[x]).wait()

    def load_sync(sb, si):
        pltpu.sync_copy(edata.at[tid, pl.ds(si * SUP, SUP)], ebufs[sb])
        pltpu.sync_copy(wdata.at[tid, pl.ds(si * SUP, SUP)], wbufs[sb])

    def multiply(sb, lc, x):
        def g_body(g, carry):
            wvec = wbufs[sb][lc, pl.ds(g * LN, LN)]
            for l in range(LN):
                wl = wvec.at[jnp.full((LN,), l, jnp.int32)].get(
                    mode="promise_in_bounds")
                e = g * LN + l
                for j in range(CH // LN):
                    rows[x][e, pl.ds(j * LN, LN)] = (
                        rows[x][e, pl.ds(j * LN, LN)] * wl)
            return carry
        lax.fori_loop(0, C // LN, g_body, 0)

    def half_body(h, carry):
        pltpu.sync_copy(table2.at[h, pl.ds(s * RPT, RPT)],
                        tab.at[pl.ds(s * RPT, RPT)])
        pltpu.sync_copy(zrows.at[pl.ds(0, RPT)], acc.at[pl.ds(s * RPT, RPT)])

        @pl.when(s == NS - 1)
        def _():
            pltpu.sync_copy(table2.at[h, pl.ds(NS * RPT, REM)],
                            tab.at[pl.ds(NS * RPT, REM)])
            pltpu.sync_copy(zrows.at[pl.ds(RPT, REM)],
                            acc.at[pl.ds(NS * RPT, REM)])

        load_sync(0, 0)
        plsc.subcore_barrier()

        for pre in range(NRING - 1):
            gather_start(0, pre, pre)

        def super_pair(u, carry2):
            for sb in range(2):
                si = u * 2 + sb
                base = si * SUP
                for lc in range(SUP):
                    d = lc % NRING
                    ch = base + lc
                    gather_wait(d)
                    if lc == 4:
                        if sb == 0:
                            load_sync(1, si + 1)
                        else:
                            @pl.when(si + 1 < NSUP)
                            def _():
                                load_sync(0, si + 1)
                    nx = (lc + NRING - 1) % SUP
                    nd = (d + NRING - 1) % NRING
                    if lc + NRING - 1 < SUP:
                        @pl.when(ch >= 1)
                        def _():
                            scatter_wait(nd)
                        gather_start(sb, nx, nd)
                    elif sb == 0:
                        scatter_wait(nd)
                        gather_start(1, nx, nd)
                    else:
                        @pl.when(si + 1 < NSUP)
                        def _():
                            scatter_wait(nd)
                            gather_start(0, nx, nd)
                    multiply(sb, lc, d)
                    scatter_start(sb, lc, d)
            return carry2

        lax.fori_loop(0, NSUP // 2, super_pair, 0)
        for x in range(NRING):
            scatter_wait(x)
        plsc.subcore_barrier()

        pltpu.sync_copy(acc.at[pl.ds(s * RPT, RPT)],
                        po.at[c, h, pl.ds(s * RPT, RPT)])

        @pl.when(s == NS - 1)
        def _():
            pltpu.sync_copy(acc.at[pl.ds(NS * RPT, REM)],
                            po.at[c, h, pl.ds(NS * RPT, REM)])

        plsc.subcore_barrier()
        return carry

    lax.fori_loop(0, 2, half_body, 0)


_sc_mesh = plsc.VectorSubcoreMesh(
    core_axis_name="c", subcore_axis_name="s", num_cores=NC, num_subcores=NS)

_sc_call = pl.kernel(
    _sc_body,
    out_type=[jax.ShapeDtypeStruct((NC, 2, V, CH), jnp.float32)],
    mesh=_sc_mesh,
    scratch_types=[
        pltpu.VMEM_SHARED((V, CH), jnp.float32),
        pltpu.VMEM_SHARED((V, CH), jnp.float32),
        pltpu.VMEM((SUP, 2, C), jnp.int32),
        pltpu.VMEM((SUP, 2, C), jnp.int32),
        pltpu.VMEM((SUP, C), jnp.float32),
        pltpu.VMEM((SUP, C), jnp.float32),
        pltpu.VMEM((C, CH), jnp.float32),
        pltpu.VMEM((C, CH), jnp.float32),
        pltpu.VMEM((C, CH), jnp.float32),
        pltpu.VMEM((C, CH), jnp.float32),
        pltpu.SemaphoreType.DMA,
        pltpu.SemaphoreType.DMA,
        pltpu.SemaphoreType.DMA,
        pltpu.SemaphoreType.DMA,
        pltpu.SemaphoreType.DMA,
        pltpu.SemaphoreType.DMA,
        pltpu.SemaphoreType.DMA,
        pltpu.SemaphoreType.DMA,
    ],
)


def _sum_body(pr, outr):
    def body(b, carry):
        r0 = b * 1000
        for h in range(2):
            outr[h, pl.ds(r0, 1000), :] = (pr[0, h, pl.ds(r0, 1000), :]
                                           + pr[1, h, pl.ds(r0, 1000), :])
        return carry
    lax.fori_loop(0, V // 1000, body, 0)


_sum_call = pl.pallas_call(
    _sum_body,
    out_shape=jax.ShapeDtypeStruct((2, V, CH), jnp.float32),
)


BLK = 500
NB = V // BLK
PBLK = BLK // 4


def _tc_body(x0r, x1r, qr, war, wbr, wcr, gr, br, outr, ys):
    def mm(b, carry):
        sm, sq = carry
        r0 = b * BLK
        yb = jnp.dot(x0r[pl.ds(r0, BLK), :], war[...],
                     preferred_element_type=jnp.float32)
        yb = yb + jnp.dot(x1r[0, pl.ds(r0, BLK), :], wbr[:CH, :],
                          preferred_element_type=jnp.float32)
        yb = yb + jnp.dot(x1r[1, pl.ds(r0, BLK), :], wbr[CH:, :],
                          preferred_element_type=jnp.float32)
        yb = yb + jnp.dot(qr[0, 0, pl.ds(r0, BLK), :]
                          + qr[1, 0, pl.ds(r0, BLK), :], wcr[:CH, :],
                          preferred_element_type=jnp.float32)
        yb = yb + jnp.dot(qr[0, 1, pl.ds(r0, BLK), :]
                          + qr[1, 1, pl.ds(r0, BLK), :], wcr[CH:, :],
                          preferred_element_type=jnp.float32)
        ys[pl.ds(r0, BLK), :] = yb
        sm = sm + jnp.sum(yb, axis=0, keepdims=True)
        sq = sq + jnp.sum(yb * yb, axis=0, keepdims=True)
        return sm, sq

    zero = jnp.zeros((1, FOUT), jnp.float32)
    sm, sq = lax.fori_loop(0, NB, mm, (zero, zero))
    mean = sm / float(V)
    var = sq / float(V) - mean * mean
    scale = gr[...] * lax.rsqrt(var + 1e-5)
    shift = br[...] - mean * scale

    def norm(b, carry):
        yb = ys[pl.ds(b * BLK, BLK), :]
        yn = jnp.maximum(yb * scale + shift, 0.0)
        pooled = jnp.mean(yn.reshape(PBLK, 4, FOUT), axis=1)
        outr[pl.ds(b * PBLK, PBLK), :] = pooled
        return carry

    lax.fori_loop(0, NB, norm, 0)


_tc_call = pl.pallas_call(
    _tc_body,
    out_shape=jax.ShapeDtypeStruct((V // 4, FOUT), jnp.float32),
    scratch_shapes=[pltpu.VMEM((V, FOUT), jnp.float32)],
)


def kernel(x, edge_index, edge_weight, weight, gamma, beta):
    x0 = x[0]
    src = edge_index[0]
    dst = edge_index[1]
    pad = EP - E
    srcp = jnp.concatenate([src, jnp.zeros((pad,), jnp.int32)])
    dstp = jnp.concatenate([dst, jnp.zeros((pad,), jnp.int32)])
    wp = jnp.concatenate([edge_weight, jnp.zeros((pad,), jnp.float32)])
    edata = jnp.stack([srcp.reshape(NC * NS, NCHUNK, C),
                       dstp.reshape(NC * NS, NCHUNK, C)], axis=2)
    wdata = wp.reshape(NC * NS, NCHUNK, C)
    x0h = jnp.stack([x0[:, :CH], x0[:, CH:]])
    zrows = jnp.zeros((RPT + REM, CH), jnp.float32)

    w0 = weight[0::KCH]
    w1 = weight[1::KCH]
    w2 = weight[2::KCH]
    wa = w0 - w2
    wb = w1
    wc = 2.0 * w2

    (p,) = _sc_call(x0h, edata, wdata, zrows)
    x1h = _sum_call(p)
    (q,) = _sc_call(x1h, edata, wdata, zrows)
    out = _tc_call(x0, x1h, q, wa, wb, wc,
                   gamma.reshape(1, FOUT), beta.reshape(1, FOUT))
    return out[None]
